# baseline (speedup 1.0000x reference)
import functools

import jax
import jax.numpy as jnp
from jax import lax
from jax.experimental import pallas as pl
from jax.experimental.pallas import tpu as pltpu
from jax.experimental.pallas import tpu_sc as plsc

_N_USERS = 25000
_N_NODES = 50000
_D = 64
_E = 800000

_NC = 2
_NS = 16
_NW = _NC * _NS
_NB = _NC * _NS
_CHUNK = 128
_E_PAD = 802816
_NCH = _E_PAD // _CHUNK
_W_CH = _NCH // _NW
_BLK = 7
_NBLK = _W_CH // _BLK
_HALF = _N_NODES // _NC
_TROWS = 1568
_SEG2 = 20
_SBIAS = 16
_SECT = _SBIAS + _BLK * _CHUNK + 144
_CP_ROWS = _HALF - 15 * _TROWS


def _part_body(packed, valsh, pp, vp, cnts,
               ie0, ie1, iv0, iv1, stc, strr, stv, cbuf,
               fills, kcnt, si0, si1, sf):
    core = lax.axis_index("c")
    sid = lax.axis_index("s")
    w = core * _NS + sid
    q0 = w * _W_CH
    ie, iv, si = (ie0, ie1), (iv0, iv1), (si0, si1)
    lanes = lax.iota(jnp.int32, 16)

    def iload(blk, p):
        qb = q0 + blk * _BLK
        pltpu.async_copy(packed.at[pl.ds(qb, _BLK)], ie[p], si[p])
        pltpu.async_copy(valsh.at[pl.ds(qb * _CHUNK, _BLK * _CHUNK)], iv[p], si[p])

    def iwait(p):
        pltpu.make_async_copy(packed.at[pl.ds(0, _BLK)], ie[p], si[p]).wait()
        pltpu.make_async_copy(valsh.at[pl.ds(0, _BLK * _CHUNK)], iv[p], si[p]).wait()

    def init_body(j, carry):
        fills[j] = jnp.int32(0)
        kcnt[j] = jnp.int32(0)
        return carry

    lax.fori_loop(0, _NB, init_body, 0)

    def route_group(eb, vb, q, g):
        sl16 = pl.ds(g * 16, 16)
        c16 = eb[q, 0, sl16]
        r16 = eb[q, 1, sl16]
        v16 = vb[pl.ds(q * _CHUNK + g * 16, 16)]
        for k in range(16):
            rot = (lanes + k) % 16
            crot = c16[rot]
            rrot = r16[rot]
            vrot = v16[rot]
            r = r16[k]
            h = jnp.where(r >= _HALF, 1, 0).astype(jnp.int32)
            lr = r - h * _HALF
            t = ((lr >> 5) * 1338) >> 16
            bkt = h * _NS + t
            f = fills[bkt]
            off = bkt * _SECT + _SBIAS + f
            stc[pl.ds(off, 16)] = crot
            strr[pl.ds(off, 16)] = rrot
            stv[pl.ds(off, 16)] = vrot
            fills[bkt] = f + 1

    def seg_slot(bkt, j):
        return (w * _NB + bkt) * _SEG2 + jnp.minimum(j, _SEG2 - 1)

    def issue_chunk(bkt, src_chunk, dst_slot):
        sl = pl.ds(bkt * _SECT + _SBIAS + src_chunk * _CHUNK, _CHUNK)
        pltpu.async_copy(stc.at[sl], pp.at[dst_slot, 0], sf)
        pltpu.async_copy(strr.at[sl], pp.at[dst_slot, 1], sf)
        pltpu.async_copy(stv.at[sl], vp.at[pl.ds(dst_slot * _CHUNK, _CHUNK)], sf)

    def wait_chunk(j, carry):
        sl0 = pl.ds(_SBIAS, _CHUNK)
        pltpu.make_async_copy(stc.at[sl0], pp.at[0, 0], sf).wait()
        pltpu.make_async_copy(strr.at[sl0], pp.at[0, 1], sf).wait()
        pltpu.make_async_copy(stv.at[sl0], vp.at[pl.ds(0, _CHUNK)], sf).wait()
        return carry

    def flush_all():
        def fb(bkt, nf):
            f = fills[bkt]
            kb = f >> 7
            kc = kcnt[bkt]

            def dmas(j, c):
                issue_chunk(bkt, j, seg_slot(bkt, kc + j))
                return c

            lax.fori_loop(0, kb, dmas, 0)
            kcnt[bkt] = kc + kb
            return nf + kb

        nf = lax.fori_loop(0, _NB, fb, jnp.int32(0))
        lax.fori_loop(0, nf, wait_chunk, 0)

        def mv(bkt, carry):
            f = fills[bkt]
            kb = f >> 7
            hb = bkt * _SECT + _SBIAS
            for i in range(_CHUNK // 16):
                sl_src = pl.ds(hb + kb * _CHUNK + i * 16, 16)
                sl_dst = pl.ds(hb + i * 16, 16)
                for ref in (stc, strr, stv):
                    ref[sl_dst] = ref[sl_src]
            fills[bkt] = f - kb * _CHUNK
            return carry

        lax.fori_loop(0, _NB, mv, 0)

    def block_pass(bi, p):
        iload(jnp.minimum(bi + 1, _NBLK - 1), 1 - p)
        iwait(p)

        def chunk_body(q, carry):
            def group_body(g, gc):
                route_group(ie[p], iv[p], q, g)
                return gc

            return lax.fori_loop(0, _CHUNK // 16, group_body, carry)

        lax.fori_loop(0, _BLK, chunk_body, 0)
        flush_all()

    iload(0, 0)

    def two_blocks(i, carry):
        block_pass(2 * i, 0)
        block_pass(2 * i + 1, 1)
        return carry

    lax.fori_loop(0, _NBLK // 2, two_blocks, 0)
    iwait(0)

    def fin(bkt, carry):
        f = fills[bkt]
        kc = kcnt[bkt]
        h = bkt >> 4
        nullrow = h * _HALF + (bkt - h * _NS) * _TROWS
        hb = bkt * _SECT + _SBIAS
        zi = jnp.zeros((16,), jnp.int32)
        zr = jnp.zeros((16,), jnp.int32) + nullrow
        zf = jnp.zeros((16,), jnp.float32)
        for i in range(_CHUNK // 16):
            sl = pl.ds(hb + f + i * 16, 16)
            stc[sl] = zi
            strr[sl] = zr
            stv[sl] = zf

        @pl.when(f > 0)
        def _():
            issue_chunk(bkt, 0, seg_slot(bkt, kc))
            lax.fori_loop(0, 1, wait_chunk, 0)

        kc = kc + jnp.where(f > 0, 1, 0)
        for i in range(_CHUNK // 16):
            sl = pl.ds(hb + i * 16, 16)
            stc[sl] = zi
            strr[sl] = zr
            stv[sl] = zf
        npad = jnp.where(kc == 0, 2, kc & 1)

        def nb(j, c):
            issue_chunk(bkt, 0, seg_slot(bkt, kc + j))
            lax.fori_loop(0, 1, wait_chunk, 0)
            return c

        lax.fori_loop(0, npad, nb, 0)
        kc = kc + npad
        cbuf[pl.ds(0, 16)] = jnp.zeros((16,), jnp.int32) + jnp.minimum(kc, _SEG2)
        pltpu.sync_copy(cbuf, cnts.at[w * _NB + bkt])
        return carry

    lax.fori_loop(0, _NB, fin, 0)


_part = functools.partial(
    pl.kernel,
    mesh=plsc.VectorSubcoreMesh(core_axis_name="c", subcore_axis_name="s"),
    compiler_params=pltpu.CompilerParams(use_tc_tiling_on_sc=False),
    out_type=(
        jax.ShapeDtypeStruct((_NW * _NB * _SEG2, 2, _CHUNK), jnp.int32),
        jax.ShapeDtypeStruct((_NW * _NB * _SEG2 * _CHUNK,), jnp.float32),
        jax.ShapeDtypeStruct((_NW * _NB, 16), jnp.int32),
    ),
    scratch_types=(
        [pltpu.VMEM((_BLK, 2, _CHUNK), jnp.int32) for _ in range(2)]
        + [pltpu.VMEM((_BLK * _CHUNK,), jnp.float32) for _ in range(2)]
        + [pltpu.VMEM((_NB * _SECT,), jnp.int32),
           pltpu.VMEM((_NB * _SECT,), jnp.int32),
           pltpu.VMEM((_NB * _SECT,), jnp.float32),
           pltpu.VMEM((16,), jnp.int32),
           pltpu.SMEM((_NB,), jnp.int32),
           pltpu.SMEM((_NB,), jnp.int32)]
        + [pltpu.SemaphoreType.DMA for _ in range(3)]
    ),
)(_part_body)


def _accum_chunk(ebuf, vbuf, gbuf, acc, tbase):
    for g in range(_CHUNK // 16):
        sl16 = pl.ds(g * 16, 16)
        lr = ebuf[1, sl16] - tbase
        vv = vbuf[sl16]
        for k in range(16):
            i = g * 16 + k
            row = lr[k]
            v = vv[k]
            for j in range(_D // 16):
                sl = pl.ds(j * 16, 16)
                acc[row, sl] = acc[row, sl] + gbuf[i, sl] * v


def _prop_body(table, pp, vp, cnts, out,
               ebuf0, ebuf1, vbuf0, vbuf1, gbuf0, gbuf1, acc, cbuf,
               se0, se1, sg0, sg1):
    core = lax.axis_index("c")
    sid = lax.axis_index("s")
    bkt = core * _NS + sid
    tbase = core * _HALF + sid * _TROWS
    ebuf, vbuf = (ebuf0, ebuf1), (vbuf0, vbuf1)
    gbuf = (gbuf0, gbuf1)
    se, sg = (se0, se1), (sg0, sg1)

    def eload(qc, b):
        pltpu.async_copy(pp.at[qc], ebuf[b], se[b])
        pltpu.async_copy(vp.at[pl.ds(qc * _CHUNK, _CHUNK)], vbuf[b], se[b])

    def ewait(b):
        pltpu.make_async_copy(pp.at[0], ebuf[b], se[b]).wait()
        pltpu.make_async_copy(vp.at[pl.ds(0, _CHUNK)], vbuf[b], se[b]).wait()

    def gather(b):
        pltpu.async_copy(table.at[ebuf[b].at[0]], gbuf[b], sg[b])

    def gwait(b):
        pltpu.make_async_copy(table.at[ebuf[b].at[0]], gbuf[b], sg[b]).wait()

    zero = jnp.zeros((16,), jnp.float32)

    def zbody(i, carry):
        for j in range(_D // 16):
            acc[i, pl.ds(j * 16, 16)] = zero
        return carry

    lax.fori_loop(0, _TROWS, zbody, 0)

    def run_segment(wk, carry):
        pltpu.sync_copy(cnts.at[wk * _NB + bkt], cbuf)
        k_seg = cbuf[pl.ds(0, 16)][0]
        seg = (wk * _NB + bkt) * _SEG2

        def q_of(c):
            return seg + jnp.minimum(c, k_seg - 1)

        eload(q_of(0), 0)
        ewait(0)
        gather(0)
        eload(q_of(1), 1)

        def chunk_pair(i, c2):
            for b in (0, 1):
                c = 2 * i + b
                nb = 1 - b
                ewait(nb)
                gather(nb)
                gwait(b)
                _accum_chunk(ebuf[b], vbuf[b], gbuf[b], acc, tbase)
                eload(q_of(c + 2), b)
            return c2

        lax.fori_loop(0, k_seg // 2, chunk_pair, 0)
        gwait(0)
        ewait(1)
        return carry

    lax.fori_loop(0, _NW, run_segment, 0)

    @pl.when(sid < _NS - 1)
    def _():
        pltpu.sync_copy(acc.at[pl.ds(0, _TROWS)], out.at[pl.ds(tbase, _TROWS)])

    @pl.when(sid == _NS - 1)
    def _():
        pltpu.sync_copy(acc.at[pl.ds(0, _CP_ROWS)], out.at[pl.ds(tbase, _CP_ROWS)])


_prop = functools.partial(
    pl.kernel,
    mesh=plsc.VectorSubcoreMesh(core_axis_name="c", subcore_axis_name="s"),
    compiler_params=pltpu.CompilerParams(use_tc_tiling_on_sc=False),
    out_type=jax.ShapeDtypeStruct((_N_NODES, _D), jnp.float32),
    scratch_types=(
        [pltpu.VMEM((2, _CHUNK), jnp.int32) for _ in range(2)]
        + [pltpu.VMEM((_CHUNK,), jnp.float32) for _ in range(2)]
        + [pltpu.VMEM((_CHUNK, _D), jnp.float32) for _ in range(2)]
        + [pltpu.VMEM((_TROWS, _D), jnp.float32),
           pltpu.VMEM((16,), jnp.int32)]
        + [pltpu.SemaphoreType.DMA for _ in range(4)]
    ),
)(_prop_body)


def _mean_body(a, b, c, d, o):
    o[...] = (a[...] + b[...] + c[...] + d[...]) * 0.25


def _mean(x0, x1, x2, x3):
    blk = (1000, _D)
    spec = pl.BlockSpec(blk, lambda i: (i, 0))
    return pl.pallas_call(
        _mean_body,
        grid=(_N_NODES // blk[0],),
        in_specs=[spec] * 4,
        out_specs=spec,
        out_shape=jax.ShapeDtypeStruct((_N_NODES, _D), jnp.float32),
    )(x0, x1, x2, x3)


def kernel(user_emb, item_emb, edge_index, edge_values):
    rows = jnp.asarray(edge_index[0], jnp.int32)
    cols = jnp.asarray(edge_index[1], jnp.int32)
    vals = edge_values.astype(jnp.float32)
    pad = _E_PAD - _E
    rows = jnp.concatenate([rows, jnp.zeros((pad,), jnp.int32)])
    cols = jnp.concatenate([cols, jnp.zeros((pad,), jnp.int32)])
    vals = jnp.concatenate([vals, jnp.zeros((pad,), jnp.float32)])
    packed = jnp.stack([cols.reshape(_NCH, _CHUNK),
                        rows.reshape(_NCH, _CHUNK)], axis=1)

    pp, vp, cnts = _part(packed, vals)
    x0 = jnp.concatenate([user_emb, item_emb], axis=0)
    x1 = _prop(x0, pp, vp, cnts)
    x2 = _prop(x1, pp, vp, cnts)
    x3 = _prop(x2, pp, vp, cnts)
    m = _mean(x0, x1, x2, x3)
    return m[:_N_USERS], m[_N_USERS:]


# speedup vs baseline: 7.5748x; 7.5748x over previous
import functools

import jax
import jax.numpy as jnp
from jax import lax
from jax.experimental import pallas as pl
from jax.experimental.pallas import tpu as pltpu
from jax.experimental.pallas import tpu_sc as plsc

_N_USERS = 25000
_N_NODES = 50000
_D = 64
_E = 800000

_NC = 2
_NS = 16
_CHUNK = 128
_E_PAD = 802816
_NCH = _E_PAD // _CHUNK
_CH_PER_TILE = _NCH // _NS
_HALF = _N_NODES // _NC
_ACC_ROWS = _HALF + 88
_ZR = 32
_CP_ROWS = 1560


def _scale_chunk(ebuf, vbuf, gbuf, rloc, base_row):
    for g in range(_CHUNK // 16):
        sl16 = pl.ds(g * 16, 16)
        r = ebuf[1, sl16]
        loc = r - base_row
        oob = (loc < 0) | (loc >= _HALF)
        rloc[sl16] = jnp.where(oob, _HALF, loc)
        vv = vbuf[sl16]
        for k in range(16):
            i = g * 16 + k
            v = vv[k]
            for j in range(_D // 16):
                sl = pl.ds(j * 16, 16)
                gbuf[i, sl] = gbuf[i, sl] * v


def _prop_body(table, packed, valsh, out,
               ebuf0, ebuf1, vbuf0, vbuf1, gbuf0, gbuf1, rloc0, rloc1,
               zbuf, acc, se0, se1, sg0, sg1, ss0, ss1):
    core = lax.axis_index("c")
    sid = lax.axis_index("s")
    base_row = core * _HALF
    ebuf, vbuf = (ebuf0, ebuf1), (vbuf0, vbuf1)
    gbuf, rloc = (gbuf0, gbuf1), (rloc0, rloc1)
    se, sg, ss = (se0, se1), (sg0, sg1), (ss0, ss1)

    q0 = sid * _CH_PER_TILE

    def eload(c, b):
        qc = jnp.minimum(q0 + c, _NCH - 1)
        pltpu.async_copy(packed.at[qc], ebuf[b], se[b])
        pltpu.async_copy(valsh.at[pl.ds(qc * _CHUNK, _CHUNK)], vbuf[b], se[b])

    def ewait(b):
        pltpu.make_async_copy(packed.at[q0], ebuf[b], se[b]).wait()
        pltpu.make_async_copy(valsh.at[pl.ds(0, _CHUNK)], vbuf[b], se[b]).wait()

    def gather(b):
        pltpu.async_copy(table.at[ebuf[b].at[0]], gbuf[b], sg[b])

    zero = jnp.zeros((16,), jnp.float32)
    for r in range(_ZR):
        for j in range(_D // 16):
            zbuf[r, pl.ds(j * 16, 16)] = zero

    def zloop(i, carry):
        pltpu.sync_copy(zbuf, acc.at[pl.ds(sid * 1568 + i * _ZR, _ZR)])
        return carry

    lax.fori_loop(0, 1568 // _ZR, zloop, 0)

    eload(0, 0)
    ewait(0)
    gather(0)
    eload(1, 1)
    plsc.subcore_barrier()

    def chunk_pair(i, carry):
        for b in (0, 1):
            c = 2 * i + b
            nb = 1 - b
            ewait(nb)

            @pl.when(c >= 1)
            def _():
                pltpu.make_async_copy(gbuf[nb], acc.at[rloc[nb]], ss[nb]).wait()

            gather(nb)
            pltpu.make_async_copy(table.at[ebuf[b].at[0]], gbuf[b], sg[b]).wait()
            _scale_chunk(ebuf[b], vbuf[b], gbuf[b], rloc[b], base_row)
            pltpu.async_copy(gbuf[b], acc.at[rloc[b]], ss[b], add=True)
            eload(c + 2, b)
        return carry

    lax.fori_loop(0, _CH_PER_TILE // 2, chunk_pair, 0)

    pltpu.make_async_copy(gbuf[1], acc.at[rloc[1]], ss[1]).wait()
    pltpu.make_async_copy(table.at[ebuf[0].at[0]], gbuf[0], sg[0]).wait()
    ewait(1)
    plsc.subcore_barrier()

    pltpu.sync_copy(acc.at[pl.ds(sid * _CP_ROWS, _CP_ROWS)],
                    out.at[pl.ds(base_row + sid * _CP_ROWS, _CP_ROWS)])

    @pl.when(sid < 5)
    def _():
        tail = _NS * _CP_ROWS + sid * 8
        pltpu.sync_copy(acc.at[pl.ds(tail, 8)],
                        out.at[pl.ds(base_row + tail, 8)])


_prop = functools.partial(
    pl.kernel,
    mesh=plsc.VectorSubcoreMesh(core_axis_name="c", subcore_axis_name="s"),
    compiler_params=pltpu.CompilerParams(use_tc_tiling_on_sc=False),
    out_type=jax.ShapeDtypeStruct((_N_NODES, _D), jnp.float32),
    scratch_types=(
        [pltpu.VMEM((2, _CHUNK), jnp.int32) for _ in range(2)]
        + [pltpu.VMEM((_CHUNK,), jnp.float32) for _ in range(2)]
        + [pltpu.VMEM((_CHUNK, _D), jnp.float32) for _ in range(2)]
        + [pltpu.VMEM((_CHUNK,), jnp.int32) for _ in range(2)]
        + [pltpu.VMEM((_ZR, _D), jnp.float32),
           pltpu.VMEM_SHARED((_ACC_ROWS, _D), jnp.float32)]
        + [pltpu.SemaphoreType.DMA for _ in range(6)]
    ),
)(_prop_body)


def _mean_body(a, b, c, d, o):
    o[...] = (a[...] + b[...] + c[...] + d[...]) * 0.25


def _mean(x0, x1, x2, x3):
    blk = (1000, _D)
    spec = pl.BlockSpec(blk, lambda i: (i, 0))
    return pl.pallas_call(
        _mean_body,
        grid=(_N_NODES // blk[0],),
        in_specs=[spec] * 4,
        out_specs=spec,
        out_shape=jax.ShapeDtypeStruct((_N_NODES, _D), jnp.float32),
    )(x0, x1, x2, x3)


def kernel(user_emb, item_emb, edge_index, edge_values):
    rows = jnp.asarray(edge_index[0], jnp.int32)
    cols = jnp.asarray(edge_index[1], jnp.int32)
    vals = edge_values.astype(jnp.float32)
    pad = _E_PAD - _E
    rows = jnp.concatenate([rows, jnp.zeros((pad,), jnp.int32)])
    cols = jnp.concatenate([cols, jnp.zeros((pad,), jnp.int32)])
    vals = jnp.concatenate([vals, jnp.zeros((pad,), jnp.float32)])
    packed = jnp.stack([cols.reshape(_NCH, _CHUNK),
                        rows.reshape(_NCH, _CHUNK)], axis=1)

    x0 = jnp.concatenate([user_emb, item_emb], axis=0)
    x1 = _prop(x0, packed, vals)
    x2 = _prop(x1, packed, vals)
    x3 = _prop(x2, packed, vals)
    m = _mean(x0, x1, x2, x3)
    return m[:_N_USERS], m[_N_USERS:]
